# Initial kernel scaffold; baseline (speedup 1.0000x reference)
import jax
import jax.numpy as jnp
from jax.experimental import pallas as pl


def kernel(x, gate_w, gate_b, w1, w2, w3):
    raise NotImplementedError("write your pallas kernel here")


# speedup vs baseline: 1.3556x; 1.3556x over previous
import jax
import jax.numpy as jnp
from jax.experimental import pallas as pl
from jax.experimental.pallas import tpu as pltpu

H = 1024
I = 4096
E = 8
N = 2048
K = 2

IB = 8
IBLK = I // IB


def _dense_moe_kernel(x_ref, gw_ref, gb_ref, w1_ref, w2_ref, w3_ref,
                      out_ref, comb_ref, xbf_ref, acc_ref):
    e = pl.program_id(0)
    i = pl.program_id(1)

    @pl.when((e == 0) & (i == 0))
    def _router():
        x = x_ref[...]
        logits = jax.lax.dot_general(
            x, gw_ref[...], (((1,), (1,)), ((), ())),
            preferred_element_type=jnp.float32) + gb_ref[...]
        m = jnp.max(logits, axis=1, keepdims=True)
        ex = jnp.exp(logits - m)
        v = ex / jnp.sum(ex, axis=1, keepdims=True)
        lane = jax.lax.broadcasted_iota(jnp.int32, (N, E), 1)
        m1 = jnp.max(v, axis=1, keepdims=True)
        e1 = jnp.min(jnp.where(v == m1, lane, E), axis=1, keepdims=True)
        sel1 = lane == e1
        vm = jnp.where(sel1, -1.0, v)
        m2 = jnp.max(vm, axis=1, keepdims=True)
        e2 = jnp.min(jnp.where(vm == m2, lane, E), axis=1, keepdims=True)
        sel2 = lane == e2
        s = m1 + m2
        comb_ref[...] = (jnp.where(sel1, m1, 0.0)
                         + jnp.where(sel2, m2, 0.0)) / s
        xbf_ref[...] = x.astype(jnp.bfloat16)
        out_ref[...] = jnp.zeros_like(out_ref)

    @pl.when(i == 0)
    def _zero_acc():
        acc_ref[...] = jnp.zeros_like(acc_ref)

    xb = xbf_ref[...]
    w1b = w1_ref[0].astype(jnp.bfloat16)
    w3b = w3_ref[0].astype(jnp.bfloat16)
    w2b = w2_ref[0].astype(jnp.bfloat16)
    h1 = jax.lax.dot_general(xb, w1b, (((1,), (1,)), ((), ())),
                             preferred_element_type=jnp.float32)
    h3 = jax.lax.dot_general(xb, w3b, (((1,), (1,)), ((), ())),
                             preferred_element_type=jnp.float32)
    hid = (h1 * jax.lax.logistic(h1) * h3).astype(jnp.bfloat16)
    eo = jax.lax.dot_general(hid, w2b, (((1,), (1,)), ((), ())),
                             preferred_element_type=jnp.float32)
    acc_ref[...] += eo

    @pl.when(i == IB - 1)
    def _combine():
        lane = jax.lax.broadcasted_iota(jnp.int32, (N, E), 1)
        c = jnp.sum(jnp.where(lane == e, comb_ref[...], 0.0),
                    axis=1, keepdims=True)
        out_ref[...] += c * acc_ref[...]


def kernel(x, gate_w, gate_b, w1, w2, w3):
    gb = gate_b.reshape(1, E)
    return pl.pallas_call(
        _dense_moe_kernel,
        grid=(E, IB),
        in_specs=[
            pl.BlockSpec((N, H), lambda e, i: (0, 0)),
            pl.BlockSpec((E, H), lambda e, i: (0, 0)),
            pl.BlockSpec((1, E), lambda e, i: (0, 0)),
            pl.BlockSpec((1, IBLK, H), lambda e, i: (e, i, 0)),
            pl.BlockSpec((1, H, IBLK), lambda e, i: (e, 0, i)),
            pl.BlockSpec((1, IBLK, H), lambda e, i: (e, i, 0)),
        ],
        out_specs=pl.BlockSpec((N, H), lambda e, i: (0, 0)),
        out_shape=jax.ShapeDtypeStruct((N, H), jnp.float32),
        scratch_shapes=[
            pltpu.VMEM((N, E), jnp.float32),
            pltpu.VMEM((N, H), jnp.bfloat16),
            pltpu.VMEM((N, H), jnp.float32),
        ],
        compiler_params=pltpu.CompilerParams(
            dimension_semantics=("arbitrary", "arbitrary")),
    )(x, gate_w, gb, w1, w2, w3)
